# baseline (speedup 1.0000x reference)
import functools

import jax
import jax.numpy as jnp
from jax import lax
from jax.experimental import pallas as pl
from jax.experimental.pallas import tpu as pltpu
from jax.experimental.pallas import tpu_sc as plsc

_VOCAB = 100000
_DIM = 1024
_B = 4
_S = 2048
_EPS = 1e-5

_NC = 2
_NS = 16
_NW = _NC * _NS
_ROWS = _B * _S
_RPW = _ROWS // _NW
_CH = 32
_NCHUNK = _RPW // _CH
_LANES = 16
_NSLICE = _DIM // _LANES


def _rsqrt16(v):
    i = lax.bitcast_convert_type(v, jnp.int32)
    y = lax.bitcast_convert_type(jnp.int32(0x5F3759DF) - (i >> 1), jnp.float32)
    for _ in range(3):
        y = y * (1.5 - 0.5 * v * y * y)
    return y


def _emb_ln_kernel(x_hbm, table_hbm, pos_hbm, gamma_hbm, beta_hbm, out_hbm,
                   idx_v, tok_v, pos_v, gamma_v, beta_v, sem):
    wid = lax.axis_index("s") * _NC + lax.axis_index("c")
    row0 = wid * _RPW
    pos0 = row0 & (_S - 1)

    pltpu.sync_copy(gamma_hbm, gamma_v)
    pltpu.sync_copy(beta_hbm, beta_v)

    zeros = jnp.zeros((_LANES,), jnp.float32)

    for c in range(_NCHUNK):
        base = pl.multiple_of(row0 + c * _CH, _CH)
        pbase = pl.multiple_of(pos0 + c * _CH, _CH)
        pltpu.sync_copy(x_hbm.at[pl.ds(base, _CH)], idx_v)
        gather = pltpu.async_copy(table_hbm.at[idx_v], tok_v, sem)
        pltpu.sync_copy(pos_hbm.at[pl.ds(pbase, _CH)], pos_v)
        gather.wait()

        def row_body(r, _):
            def p1(jo, carry):
                s, q = carry
                for k in range(4):
                    off = jo * 64 + k * 16
                    e = tok_v[r, pl.ds(off, _LANES)] + pos_v[r, pl.ds(off, _LANES)]
                    tok_v[r, pl.ds(off, _LANES)] = e
                    s = s + e
                    q = q + e * e
                return s, q

            s, q = lax.fori_loop(0, _NSLICE // 4, p1, (zeros, zeros))
            mean = jnp.sum(s) * (1.0 / _DIM)
            var = jnp.sum(q) * (1.0 / _DIM) - mean * mean
            rstd = _rsqrt16(jnp.full((_LANES,), var + _EPS, jnp.float32))
            mean_b = jnp.full((_LANES,), mean, jnp.float32)

            def p2(jo, carry):
                for k in range(4):
                    off = jo * 64 + k * 16
                    e = tok_v[r, pl.ds(off, _LANES)]
                    g = gamma_v[pl.ds(off, _LANES)]
                    bb = beta_v[pl.ds(off, _LANES)]
                    tok_v[r, pl.ds(off, _LANES)] = (e - mean_b) * rstd * g + bb
                return carry

            lax.fori_loop(0, _NSLICE // 4, p2, 0)
            return 0

        lax.fori_loop(0, _CH, row_body, 0)
        pltpu.sync_copy(tok_v, out_hbm.at[pl.ds(base, _CH)])


@jax.jit
def _run(x_flat, input_emb, pos_emb, gamma, beta):
    mesh = plsc.VectorSubcoreMesh(core_axis_name="c", subcore_axis_name="s")
    k = functools.partial(
        pl.kernel,
        mesh=mesh,
        out_type=jax.ShapeDtypeStruct((_ROWS, _DIM), jnp.float32),
        compiler_params=pltpu.CompilerParams(needs_layout_passes=False),
        scratch_types=[
            pltpu.VMEM((_CH,), jnp.int32),
            pltpu.VMEM((_CH, _DIM), jnp.float32),
            pltpu.VMEM((_CH, _DIM), jnp.float32),
            pltpu.VMEM((_DIM,), jnp.float32),
            pltpu.VMEM((_DIM,), jnp.float32),
            pltpu.SemaphoreType.DMA,
        ],
    )(_emb_ln_kernel)
    return k(x_flat, input_emb, pos_emb, gamma, beta)


def kernel(x, input_emb, pos_emb, gamma, beta):
    x_flat = x.reshape(-1).astype(jnp.int32)
    out = _run(x_flat, input_emb, pos_emb, gamma, beta)
    return out.reshape(_B, _S, _DIM)


# speedup vs baseline: 1.3688x; 1.3688x over previous
import functools

import jax
import jax.numpy as jnp
from jax import lax
from jax.experimental import pallas as pl
from jax.experimental.pallas import tpu as pltpu
from jax.experimental.pallas import tpu_sc as plsc

_VOCAB = 100000
_DIM = 1024
_B = 4
_S = 2048
_EPS = 1e-5

_NC = 2
_NS = 16
_NW = _NC * _NS
_ROWS = _B * _S
_PPW = _S // _NW
_P = 8
_CH = _P * _B
_NCHUNK = _PPW // _P
_LANES = 16
_NSLICE = _DIM // _LANES
_GRP = 8


def _rsqrt16(v):
    i = lax.bitcast_convert_type(v, jnp.int32)
    y = lax.bitcast_convert_type(jnp.int32(0x5F3759DF) - (i >> 1), jnp.float32)
    for _ in range(3):
        y = y * (1.5 - 0.5 * v * y * y)
    return y


def _emb_ln_kernel(x_hbm, table_hbm, pos_hbm, gamma_hbm, beta_hbm, out_hbm,
                   idx_v, tok_v, pos_v, gamma_v, beta_v, meanb, rstdb,
                   gsem, psem, isem, osem):
    wid = lax.axis_index("s") * _NC + lax.axis_index("c")
    pos0 = pl.multiple_of(wid * _PPW, _PPW)

    pltpu.sync_copy(gamma_hbm, gamma_v)
    pltpu.sync_copy(beta_hbm, beta_v)

    zeros = jnp.zeros((_LANES,), jnp.float32)

    def issue_idx(c):
        bb = c % 2
        cps = []
        for b in range(_B):
            src = pl.multiple_of(b * _S + pos0 + c * _P, _P)
            cps.append(pltpu.async_copy(
                x_hbm.at[pl.ds(src, _P)], idx_v.at[bb, pl.ds(b * _P, _P)],
                isem[bb]))
        return cps

    def issue_gather(c):
        bb = c % 2
        return pltpu.async_copy(table_hbm.at[idx_v.at[bb]], tok_v.at[bb],
                                gsem[bb])

    def issue_pos(c):
        bb = c % 2
        src = pl.multiple_of(pos0 + c * _P, _P)
        return pltpu.async_copy(pos_hbm.at[pl.ds(src, _P)], pos_v.at[bb],
                                psem[bb])

    def issue_store(c):
        bb = c % 2
        cps = []
        for b in range(_B):
            dst = pl.multiple_of(b * _S + pos0 + c * _P, _P)
            cps.append(pltpu.async_copy(
                tok_v.at[bb, pl.ds(b * _P, _P)], out_hbm.at[pl.ds(dst, _P)],
                osem[bb]))
        return cps

    def compute(c):
        bb = c % 2
        tok = tok_v.at[bb]
        pos = pos_v.at[bb]

        def row_body(r, _):
            def p1(jo, carry):
                s, q = carry
                for k in range(16):
                    off = jo * 256 + k * 16
                    e = tok[r, pl.ds(off, _LANES)] + pos[r & (_P - 1), pl.ds(off, _LANES)]
                    tok[r, pl.ds(off, _LANES)] = e
                    s = s + e
                    q = q + e * e
                return s, q

            s, q = lax.fori_loop(0, _NSLICE // 16, p1, (zeros, zeros))
            mean = jnp.sum(s) * (1.0 / _DIM)
            var = jnp.sum(q) * (1.0 / _DIM) - mean * mean
            rstd = _rsqrt16(jnp.full((_LANES,), var + _EPS, jnp.float32))
            meanb[r] = jnp.full((_LANES,), mean, jnp.float32)
            rstdb[r] = rstd
            return 0

        lax.fori_loop(0, _CH, row_body, 0)

        for g0 in range(0, _CH, _GRP):
            ss = [rstdb[g0 + k] for k in range(_GRP)]
            tt = [meanb[g0 + k] * ss[k] for k in range(_GRP)]

            def p2(j, carry):
                off = j * 16
                gm = gamma_v[pl.ds(off, _LANES)]
                bt = beta_v[pl.ds(off, _LANES)]
                for k in range(_GRP):
                    e = tok[g0 + k, pl.ds(off, _LANES)]
                    tok[g0 + k, pl.ds(off, _LANES)] = (e * ss[k] - tt[k]) * gm + bt
                return carry

            lax.fori_loop(0, _NSLICE, p2, 0)

    idx_cps = [None, None]
    store_cps = [None, None]
    gather_cp = [None, None]
    pos_cp = [None, None]

    idx_cps[0] = issue_idx(0)
    for cp in idx_cps[0]:
        cp.wait()
    gather_cp[0] = issue_gather(0)
    pos_cp[0] = issue_pos(0)
    idx_cps[1] = issue_idx(1)

    for c in range(_NCHUNK):
        bb = c % 2
        nb = (c + 1) % 2
        if c + 1 < _NCHUNK:
            if store_cps[nb] is not None:
                for cp in store_cps[nb]:
                    cp.wait()
                store_cps[nb] = None
            for cp in idx_cps[nb]:
                cp.wait()
            gather_cp[nb] = issue_gather(c + 1)
            pos_cp[nb] = issue_pos(c + 1)
        gather_cp[bb].wait()
        pos_cp[bb].wait()
        if c + 2 < _NCHUNK:
            idx_cps[bb] = issue_idx(c + 2)
        compute(c)
        store_cps[bb] = issue_store(c)

    for cps in store_cps:
        if cps is not None:
            for cp in cps:
                cp.wait()


@jax.jit
def _run(x_flat, input_emb, pos_emb, gamma, beta):
    mesh = plsc.VectorSubcoreMesh(core_axis_name="c", subcore_axis_name="s")
    k = functools.partial(
        pl.kernel,
        mesh=mesh,
        out_type=jax.ShapeDtypeStruct((_ROWS, _DIM), jnp.float32),
        compiler_params=pltpu.CompilerParams(needs_layout_passes=False),
        scratch_types=[
            pltpu.VMEM((2, _CH), jnp.int32),
            pltpu.VMEM((2, _CH, _DIM), jnp.float32),
            pltpu.VMEM((2, _P, _DIM), jnp.float32),
            pltpu.VMEM((_DIM,), jnp.float32),
            pltpu.VMEM((_DIM,), jnp.float32),
            pltpu.VMEM((_CH, _LANES), jnp.float32),
            pltpu.VMEM((_CH, _LANES), jnp.float32),
            [pltpu.SemaphoreType.DMA] * 2,
            [pltpu.SemaphoreType.DMA] * 2,
            [pltpu.SemaphoreType.DMA] * 2,
            [pltpu.SemaphoreType.DMA] * 2,
        ],
    )(_emb_ln_kernel)
    return k(x_flat, input_emb, pos_emb, gamma, beta)


def kernel(x, input_emb, pos_emb, gamma, beta):
    x_flat = x.reshape(-1).astype(jnp.int32)
    out = _run(x_flat, input_emb, pos_emb, gamma, beta)
    return out.reshape(_B, _S, _DIM)
